# Initial kernel scaffold; baseline (speedup 1.0000x reference)
import jax
import jax.numpy as jnp
from jax.experimental import pallas as pl


def kernel(x, w_gate, b_gate, expert_biases):
    raise NotImplementedError("write your pallas kernel here")


# speedup vs baseline: 5.0945x; 5.0945x over previous
import jax
import jax.numpy as jnp
from jax import lax
from jax.experimental import pallas as pl
from jax.experimental.pallas import tpu as pltpu

_K = 8
_T_BLOCK = 512


def _router_body(x_ref, w_ref, bias_ref, out_ref):
    logits = jnp.dot(x_ref[...], w_ref[...], preferred_element_type=jnp.float32)
    logits = logits + bias_ref[...]
    e_dim = logits.shape[-1]
    iota = lax.broadcasted_iota(jnp.int32, logits.shape, 1)
    rowmax = jnp.max(logits, axis=-1, keepdims=True)
    work = logits
    sel = jnp.zeros(logits.shape, dtype=jnp.bool_)
    for _ in range(_K):
        m = jnp.max(work, axis=-1, keepdims=True)
        idx = jnp.min(jnp.where(work == m, iota, e_dim), axis=-1, keepdims=True)
        hit = iota == idx
        sel = jnp.logical_or(sel, hit)
        work = jnp.where(hit, -jnp.inf, work)
    ew = jnp.where(sel, jnp.exp(logits - rowmax), 0.0)
    out_ref[...] = ew / jnp.sum(ew, axis=-1, keepdims=True)


def kernel(x, w_gate, b_gate, expert_biases):
    t_dim, d_dim = x.shape
    e_dim = w_gate.shape[1]
    bias = (b_gate + expert_biases).reshape(1, e_dim).astype(jnp.float32)
    return pl.pallas_call(
        _router_body,
        grid=(t_dim // _T_BLOCK,),
        in_specs=[
            pl.BlockSpec((_T_BLOCK, d_dim), lambda i: (i, 0)),
            pl.BlockSpec((d_dim, e_dim), lambda i: (0, 0)),
            pl.BlockSpec((1, e_dim), lambda i: (0, 0)),
        ],
        out_specs=pl.BlockSpec((_T_BLOCK, e_dim), lambda i: (i, 0)),
        out_shape=jax.ShapeDtypeStruct((t_dim, e_dim), jnp.float32),
        compiler_params=pltpu.CompilerParams(
            dimension_semantics=("arbitrary",),
        ),
    )(x, w_gate, bias)


# speedup vs baseline: 6.2635x; 1.2294x over previous
import jax
import jax.numpy as jnp
from jax import lax
from jax.experimental import pallas as pl
from jax.experimental.pallas import tpu as pltpu

_K = 8
_T_BLOCK = 512


def _router_body(x_ref, w_ref, bias_ref, out_ref):
    logits = jnp.dot(x_ref[...], w_ref[...], preferred_element_type=jnp.float32)
    logits = logits + bias_ref[...]
    rowmax = jnp.max(logits, axis=-1, keepdims=True)
    work = jnp.where(logits == rowmax, -jnp.inf, logits)
    for _ in range(_K - 2):
        m = jnp.max(work, axis=-1, keepdims=True)
        work = jnp.where(work == m, -jnp.inf, work)
    thresh = jnp.max(work, axis=-1, keepdims=True)
    ew = jnp.where(logits >= thresh, jnp.exp(logits - rowmax), 0.0)
    out_ref[...] = ew / jnp.sum(ew, axis=-1, keepdims=True)


def kernel(x, w_gate, b_gate, expert_biases):
    t_dim, d_dim = x.shape
    e_dim = w_gate.shape[1]
    bias = (b_gate + expert_biases).reshape(1, e_dim).astype(jnp.float32)
    return pl.pallas_call(
        _router_body,
        grid=(t_dim // _T_BLOCK,),
        in_specs=[
            pl.BlockSpec((_T_BLOCK, d_dim), lambda i: (i, 0)),
            pl.BlockSpec((d_dim, e_dim), lambda i: (0, 0)),
            pl.BlockSpec((1, e_dim), lambda i: (0, 0)),
        ],
        out_specs=pl.BlockSpec((_T_BLOCK, e_dim), lambda i: (i, 0)),
        out_shape=jax.ShapeDtypeStruct((t_dim, e_dim), jnp.float32),
        compiler_params=pltpu.CompilerParams(
            dimension_semantics=("arbitrary",),
        ),
    )(x, w_gate, bias)


# speedup vs baseline: 6.7113x; 1.0715x over previous
import jax
import jax.numpy as jnp
from jax import lax
from jax.experimental import pallas as pl
from jax.experimental.pallas import tpu as pltpu

_K = 8
_T_BLOCK = 1024


def _router_body(x_ref, w_ref, bias_ref, out_ref):
    logits = jnp.dot(x_ref[...], w_ref[...], preferred_element_type=jnp.float32)
    logits = logits + bias_ref[...]
    rowmax = jnp.max(logits, axis=-1, keepdims=True)
    work = jnp.where(logits == rowmax, -jnp.inf, logits)
    for _ in range(_K - 2):
        m = jnp.max(work, axis=-1, keepdims=True)
        work = jnp.where(work == m, -jnp.inf, work)
    thresh = jnp.max(work, axis=-1, keepdims=True)
    ew = jnp.where(logits >= thresh, jnp.exp(logits - rowmax), 0.0)
    out_ref[...] = ew / jnp.sum(ew, axis=-1, keepdims=True)


def kernel(x, w_gate, b_gate, expert_biases):
    t_dim, d_dim = x.shape
    e_dim = w_gate.shape[1]
    bias = (b_gate + expert_biases).reshape(1, e_dim).astype(jnp.float32)
    return pl.pallas_call(
        _router_body,
        grid=(t_dim // _T_BLOCK,),
        in_specs=[
            pl.BlockSpec((_T_BLOCK, d_dim), lambda i: (i, 0)),
            pl.BlockSpec((d_dim, e_dim), lambda i: (0, 0)),
            pl.BlockSpec((1, e_dim), lambda i: (0, 0)),
        ],
        out_specs=pl.BlockSpec((_T_BLOCK, e_dim), lambda i: (i, 0)),
        out_shape=jax.ShapeDtypeStruct((t_dim, e_dim), jnp.float32),
        compiler_params=pltpu.CompilerParams(
            dimension_semantics=("arbitrary",),
        ),
    )(x, w_gate, bias)
